# baseline (speedup 1.0000x reference)
import functools

import jax
import jax.numpy as jnp
from jax import lax
from jax.experimental import pallas as pl
from jax.experimental.pallas import tpu as pltpu
from jax.experimental.pallas import tpu_sc as plsc

NC = 2
NS = 16
NW = NC * NS

BATCH = 16384
D = 128
B_PER_W = BATCH // NW
CHUNK = 32
NCHUNK = B_PER_W // CHUNK
GROUPS = CHUNK // 16
NBUF = 8


def _shuf(x, lane, s):
    return x.at[jnp.bitwise_xor(lane, s)].get(mode="promise_in_bounds")


def _combine(a, b, lane, s):
    m = (lane & s) == 0
    return (jnp.where(m, a, _shuf(b, lane, s))
            + jnp.where(m, _shuf(a, lane, s), b))


def _pack4_dots(ub, ib, r0, lane):
    accs = []
    for k in range(4):
        r = r0 + k
        ps = [ub[r, pl.ds(j * 16, 16)] * ib[r, pl.ds(j * 16, 16)]
              for j in range(D // 16)]
        accs.append(((ps[0] + ps[1]) + (ps[2] + ps[3]))
                    + ((ps[4] + ps[5]) + (ps[6] + ps[7])))
    t0 = _combine(accs[0], accs[1], lane, 1)
    t1 = _combine(accs[2], accs[3], lane, 1)
    u = _combine(t0, t1, lane, 2)
    u = u + _shuf(u, lane, 4)
    u = u + _shuf(u, lane, 8)
    return u


def _sc_body(uidx_hbm, iidx_hbm, u2e_hbm, i2e_hbm, out_hbm,
             uidx_v, iidx_v, u_rows, i_rows, out_v, sem_u, sem_i):
    wid = lax.axis_index("s") * NC + lax.axis_index("c")
    base = wid * B_PER_W

    cu0 = pltpu.async_copy(uidx_hbm.at[wid], uidx_v, sem_u)
    ci0 = pltpu.async_copy(iidx_hbm.at[wid], iidx_v, sem_i)
    cu0.wait()
    ci0.wait()

    lane = lax.iota(jnp.int32, 16)

    def idx_slice(v, c):
        cpr = 128 // CHUNK
        return v.at[c // cpr, pl.ds((c % cpr) * CHUNK, CHUNK)]

    def issue(c, buf):
        pltpu.async_copy(u2e_hbm.at[idx_slice(uidx_v, c)], u_rows.at[buf],
                         sem_u)
        pltpu.async_copy(i2e_hbm.at[idx_slice(iidx_v, c)], i_rows.at[buf],
                         sem_i)

    def drain(buf):
        pltpu.make_async_copy(u2e_hbm.at[idx_slice(uidx_v, 0)],
                              u_rows.at[buf], sem_u).wait()
        pltpu.make_async_copy(i2e_hbm.at[idx_slice(iidx_v, 0)],
                              i_rows.at[buf], sem_i).wait()

    for b in range(NBUF):
        issue(b, b)

    def compute_chunk(c, buf):
        ub = u_rows.at[buf]
        ib = i_rows.at[buf]

        def gbody(g, _):
            def rbody(i, res):
                r = g * 16 + i
                acc = ub[r, pl.ds(0, 16)] * ib[r, pl.ds(0, 16)]
                for j in range(1, D // 16):
                    acc += (ub[r, pl.ds(j * 16, 16)]
                            * ib[r, pl.ds(j * 16, 16)])
                for s in (8, 4, 2, 1):
                    acc = acc + _shuf(acc, lane, s)
                return jnp.where(lane == i, acc, res)

            res = lax.fori_loop(0, 16, rbody, jnp.zeros((16,), jnp.float32))
            out_v[pl.ds(c * CHUNK + g * 16, 16)] = (
                1.0 / (1.0 + jnp.exp(-res)))
            return 0

        lax.fori_loop(0, GROUPS, gbody, 0)

    def kbody(c, _):
        b = c & (NBUF - 1)
        drain(b)
        compute_chunk(c, b)

        @pl.when(c + NBUF < NCHUNK)
        def _issue_next():
            issue(c + NBUF, b)
        return 0

    lax.fori_loop(0, NCHUNK, kbody, 0)

    pltpu.sync_copy(out_v, out_hbm.at[pl.ds(base, B_PER_W)])


@jax.jit
def _run(uidx, iidx, u2e, i2e):
    mesh = plsc.VectorSubcoreMesh(core_axis_name="c", subcore_axis_name="s")
    f = pl.kernel(
        _sc_body,
        mesh=mesh,
        out_type=jax.ShapeDtypeStruct((BATCH,), jnp.float32),
        scratch_types=[
            pltpu.VMEM((B_PER_W // 128, 128), jnp.int32),
            pltpu.VMEM((B_PER_W // 128, 128), jnp.int32),
            pltpu.VMEM((NBUF, CHUNK, D), jnp.float32),
            pltpu.VMEM((NBUF, CHUNK, D), jnp.float32),
            pltpu.VMEM((B_PER_W,), jnp.float32),
            pltpu.SemaphoreType.DMA,
            pltpu.SemaphoreType.DMA,
        ],
    )
    return f(uidx, iidx, u2e, i2e)


def kernel(user_inputs, u_item_inputs, u2e, i2e):
    uidx = user_inputs.reshape(NW, B_PER_W // 128, 128)
    iidx = u_item_inputs.reshape(NW, B_PER_W // 128, 128)
    return _run(uidx, iidx, u2e, i2e)


# speedup vs baseline: 1.0025x; 1.0025x over previous
import functools

import jax
import jax.numpy as jnp
from jax import lax
from jax.experimental import pallas as pl
from jax.experimental.pallas import tpu as pltpu
from jax.experimental.pallas import tpu_sc as plsc

NC = 2
NS = 16
NW = NC * NS

BATCH = 16384
D = 128
B_PER_W = BATCH // NW
CHUNK = 64
NCHUNK = B_PER_W // CHUNK
GROUPS = CHUNK // 16
NBUF = 4


def _shuf(x, lane, s):
    return x.at[jnp.bitwise_xor(lane, s)].get(mode="promise_in_bounds")


def _combine(a, b, lane, s):
    m = (lane & s) == 0
    return (jnp.where(m, a, _shuf(b, lane, s))
            + jnp.where(m, _shuf(a, lane, s), b))


def _pack4_dots(ub, ib, r0, lane):
    accs = []
    for k in range(4):
        r = r0 + k
        ps = [ub[r, pl.ds(j * 16, 16)] * ib[r, pl.ds(j * 16, 16)]
              for j in range(D // 16)]
        accs.append(((ps[0] + ps[1]) + (ps[2] + ps[3]))
                    + ((ps[4] + ps[5]) + (ps[6] + ps[7])))
    t0 = _combine(accs[0], accs[1], lane, 1)
    t1 = _combine(accs[2], accs[3], lane, 1)
    u = _combine(t0, t1, lane, 2)
    u = u + _shuf(u, lane, 4)
    u = u + _shuf(u, lane, 8)
    return u


def _sc_body(uidx_hbm, iidx_hbm, u2e_hbm, i2e_hbm, out_hbm,
             uidx_v, iidx_v, u_rows, i_rows, out_v, sem_u, sem_i, sem_o):
    wid = lax.axis_index("s") * NC + lax.axis_index("c")
    base = wid * B_PER_W

    cu0 = pltpu.async_copy(uidx_hbm.at[wid], uidx_v, sem_u)
    ci0 = pltpu.async_copy(iidx_hbm.at[wid], iidx_v, sem_i)
    cu0.wait()
    ci0.wait()

    lane = lax.iota(jnp.int32, 16)

    def idx_slice(v, c):
        cpr = 128 // CHUNK
        return v.at[c // cpr, pl.ds((c % cpr) * CHUNK, CHUNK)]

    def issue(c, buf):
        pltpu.async_copy(u2e_hbm.at[idx_slice(uidx_v, c)], u_rows.at[buf],
                         sem_u)
        pltpu.async_copy(i2e_hbm.at[idx_slice(iidx_v, c)], i_rows.at[buf],
                         sem_i)

    def drain(buf):
        pltpu.make_async_copy(u2e_hbm.at[idx_slice(uidx_v, 0)],
                              u_rows.at[buf], sem_u).wait()
        pltpu.make_async_copy(i2e_hbm.at[idx_slice(iidx_v, 0)],
                              i_rows.at[buf], sem_i).wait()

    for b in range(NBUF):
        issue(b, b)

    def compute_chunk(c, buf):
        ub = u_rows.at[buf]
        ib = i_rows.at[buf]

        def gbody(g, _):
            def rbody(i, res):
                r = g * 16 + i
                acc = ub[r, pl.ds(0, 16)] * ib[r, pl.ds(0, 16)]
                for j in range(1, D // 16):
                    acc += (ub[r, pl.ds(j * 16, 16)]
                            * ib[r, pl.ds(j * 16, 16)])
                for s in (8, 4, 2, 1):
                    acc = acc + _shuf(acc, lane, s)
                return jnp.where(lane == i, acc, res)

            res = lax.fori_loop(0, 16, rbody, jnp.zeros((16,), jnp.float32))
            out_v[pl.ds(c * CHUNK + g * 16, 16)] = (
                1.0 / (1.0 + jnp.exp(-res)))
            return 0

        lax.fori_loop(0, GROUPS, gbody, 0)

    def kbody(c, _):
        b = c & (NBUF - 1)
        drain(b)
        compute_chunk(c, b)

        @pl.when(c + NBUF < NCHUNK)
        def _issue_next():
            issue(c + NBUF, b)

        pltpu.async_copy(out_v.at[pl.ds(c * CHUNK, CHUNK)],
                         out_hbm.at[pl.ds(base + c * CHUNK, CHUNK)], sem_o)
        return 0

    lax.fori_loop(0, NCHUNK, kbody, 0)

    for c in range(NCHUNK):
        pltpu.make_async_copy(out_v.at[pl.ds(0, CHUNK)],
                              out_hbm.at[pl.ds(base, CHUNK)], sem_o).wait()


@jax.jit
def _run(uidx, iidx, u2e, i2e):
    mesh = plsc.VectorSubcoreMesh(core_axis_name="c", subcore_axis_name="s")
    f = pl.kernel(
        _sc_body,
        mesh=mesh,
        out_type=jax.ShapeDtypeStruct((BATCH,), jnp.float32),
        scratch_types=[
            pltpu.VMEM((B_PER_W // 128, 128), jnp.int32),
            pltpu.VMEM((B_PER_W // 128, 128), jnp.int32),
            pltpu.VMEM((NBUF, CHUNK, D), jnp.float32),
            pltpu.VMEM((NBUF, CHUNK, D), jnp.float32),
            pltpu.VMEM((B_PER_W,), jnp.float32),
            pltpu.SemaphoreType.DMA,
            pltpu.SemaphoreType.DMA,
            pltpu.SemaphoreType.DMA,
        ],
    )
    return f(uidx, iidx, u2e, i2e)


def kernel(user_inputs, u_item_inputs, u2e, i2e):
    uidx = user_inputs.reshape(NW, B_PER_W // 128, 128)
    iidx = u_item_inputs.reshape(NW, B_PER_W // 128, 128)
    return _run(uidx, iidx, u2e, i2e)


# speedup vs baseline: 1.0070x; 1.0045x over previous
import jax
import jax.numpy as jnp
from jax import lax
from jax.experimental import pallas as pl
from jax.experimental.pallas import tpu as pltpu
from jax.experimental.pallas import tpu_sc as plsc

NC = 2
NS = 16
NW = NC * NS

BATCH = 16384
D = 128
B_PER_W = BATCH // NW
CHUNK = 64
NCHUNK = B_PER_W // CHUNK
GROUPS = CHUNK // 16
NBUF = 4


def _shuf(x, lane, s):
    return x.at[jnp.bitwise_xor(lane, s)].get(mode="promise_in_bounds")


def _sc_body(uidx_hbm, iidx_hbm, u2e_hbm, i2e_hbm, out_hbm,
             uidx_v, iidx_v, u_rows, i_rows, out_v, sem_u, sem_i):
    wid = lax.axis_index("s") * NC + lax.axis_index("c")
    base = wid * B_PER_W

    cu0 = pltpu.async_copy(uidx_hbm.at[wid], uidx_v, sem_u)
    ci0 = pltpu.async_copy(iidx_hbm.at[wid], iidx_v, sem_i)
    cu0.wait()
    ci0.wait()

    lane = lax.iota(jnp.int32, 16)

    def idx_slice(v, c):
        cpr = 128 // CHUNK
        return v.at[c // cpr, pl.ds((c % cpr) * CHUNK, CHUNK)]

    def issue(c, buf):
        pltpu.async_copy(u2e_hbm.at[idx_slice(uidx_v, c)], u_rows.at[buf],
                         sem_u)
        pltpu.async_copy(i2e_hbm.at[idx_slice(iidx_v, c)], i_rows.at[buf],
                         sem_i)

    def drain(buf):
        pltpu.make_async_copy(u2e_hbm.at[idx_slice(uidx_v, 0)],
                              u_rows.at[buf], sem_u).wait()
        pltpu.make_async_copy(i2e_hbm.at[idx_slice(iidx_v, 0)],
                              i_rows.at[buf], sem_i).wait()

    for b in range(NBUF):
        issue(b, b)

    def compute_chunk(c, buf):
        ub = u_rows.at[buf]
        ib = i_rows.at[buf]

        def gbody(g, _):
            def rbody(i, res):
                r = g * 16 + i
                acc = ub[r, pl.ds(0, 16)] * ib[r, pl.ds(0, 16)]
                for j in range(1, D // 16):
                    acc += (ub[r, pl.ds(j * 16, 16)]
                            * ib[r, pl.ds(j * 16, 16)])
                for s in (8, 4, 2, 1):
                    acc = acc + _shuf(acc, lane, s)
                return jnp.where(lane == i, acc, res)

            res = lax.fori_loop(0, 16, rbody, jnp.zeros((16,), jnp.float32))
            out_v[pl.ds(c * CHUNK + g * 16, 16)] = (
                1.0 / (1.0 + jnp.exp(-res)))
            return 0

        lax.fori_loop(0, GROUPS, gbody, 0)

    def kbody(c, _):
        b = c & (NBUF - 1)
        drain(b)
        compute_chunk(c, b)

        @pl.when(c + NBUF < NCHUNK)
        def _issue_next():
            issue(c + NBUF, b)
        return 0

    lax.fori_loop(0, NCHUNK, kbody, 0)

    pltpu.sync_copy(out_v, out_hbm.at[pl.ds(base, B_PER_W)])


@jax.jit
def _run(uidx, iidx, u2e, i2e):
    mesh = plsc.VectorSubcoreMesh(core_axis_name="c", subcore_axis_name="s")
    f = pl.kernel(
        _sc_body,
        mesh=mesh,
        out_type=jax.ShapeDtypeStruct((BATCH,), jnp.float32),
        scratch_types=[
            pltpu.VMEM((B_PER_W // 128, 128), jnp.int32),
            pltpu.VMEM((B_PER_W // 128, 128), jnp.int32),
            pltpu.VMEM((NBUF, CHUNK, D), jnp.float32),
            pltpu.VMEM((NBUF, CHUNK, D), jnp.float32),
            pltpu.VMEM((B_PER_W,), jnp.float32),
            pltpu.SemaphoreType.DMA,
            pltpu.SemaphoreType.DMA,
        ],
    )
    return f(uidx, iidx, u2e, i2e)


def kernel(user_inputs, u_item_inputs, u2e, i2e):
    uidx = user_inputs.reshape(NW, B_PER_W // 128, 128)
    iidx = u_item_inputs.reshape(NW, B_PER_W // 128, 128)
    return _run(uidx, iidx, u2e, i2e)
